# baseline (speedup 1.0000x reference)
import functools

import jax
import jax.numpy as jnp
from jax import lax
from jax.experimental import pallas as pl
from jax.experimental.pallas import tpu as pltpu
from jax.experimental.pallas import tpu_sc as plsc


def _make_gather(N: int, V: int, D: int):
    info = plsc.get_sparse_core_info()
    NC, NS = info.num_cores, info.num_subcores
    NW = NC * NS
    rows_per_w = N // NW
    C = 32
    NBUF = 2
    n_chunks = rows_per_w // C
    assert n_chunks % NBUF == 0 and n_chunks >= 2 * NBUF

    mesh = plsc.VectorSubcoreMesh(core_axis_name="c", subcore_axis_name="s")

    @functools.partial(
        pl.kernel,
        mesh=mesh,
        out_type=jax.ShapeDtypeStruct((N, D), jnp.float32),
        scratch_types=[
            pltpu.VMEM((rows_per_w,), jnp.int32),
            pltpu.VMEM((NBUF, C, D), jnp.float32),
            pltpu.SemaphoreType.DMA((NBUF,)),
            pltpu.SemaphoreType.DMA((NBUF,)),
        ],
    )
    def gather_kernel(table_hbm, idx_hbm, out_hbm, idx_v, rows_v, gsem, wsem):
        wid = lax.axis_index("s") * NC + lax.axis_index("c")
        base = wid * rows_per_w

        def gather_start(j, b):
            pltpu.async_copy(
                table_hbm.at[idx_v.at[pl.ds(j * C, C)]], rows_v.at[b],
                gsem.at[b])

        def gather_wait(j, b):
            pltpu.make_async_copy(
                table_hbm.at[idx_v.at[pl.ds(j * C, C)]], rows_v.at[b],
                gsem.at[b]).wait()

        def write_start(j, b):
            pltpu.async_copy(
                rows_v.at[b], out_hbm.at[pl.ds(base + j * C, C)], wsem.at[b])

        def write_wait(j, b):
            pltpu.make_async_copy(
                rows_v.at[b], out_hbm.at[pl.ds(base + j * C, C)],
                wsem.at[b]).wait()

        pltpu.sync_copy(idx_hbm.at[pl.ds(base, rows_per_w)], idx_v)
        for b in range(NBUF):
            gather_start(b, b)
        def round_body(g):
            for b in range(NBUF):
                gather_wait(g + b, b)
                write_start(g + b, b)
            for b in range(NBUF):
                write_wait(g + b, b)
                gather_start(g + b + NBUF, b)
        pl.loop(0, n_chunks - NBUF, step=NBUF)(round_body)
        for b in range(NBUF):
            j = n_chunks - NBUF + b
            gather_wait(j, b)
            write_start(j, b)
        for b in range(NBUF):
            j = n_chunks - NBUF + b
            write_wait(j, b)

    return gather_kernel


def kernel(input_ids, word_embeddings):
    B, S = input_ids.shape
    V, D = word_embeddings.shape
    N = B * S
    ids_flat = input_ids.reshape(N).astype(jnp.int32)
    out = _make_gather(N, V, D)(word_embeddings, ids_flat)
    return out.reshape(B, S, D)


# speedup vs baseline: 1.0238x; 1.0238x over previous
import functools

import jax
import jax.numpy as jnp
from jax import lax
from jax.experimental import pallas as pl
from jax.experimental.pallas import tpu as pltpu
from jax.experimental.pallas import tpu_sc as plsc


def _make_gather(B: int, S: int, V: int, D: int):
    info = plsc.get_sparse_core_info()
    NC, NS = info.num_cores, info.num_subcores
    NW = NC * NS
    N = B * S
    rows_per_w = N // NW
    C = 16
    NBUF = 4
    n_chunks = rows_per_w // C
    assert n_chunks % NBUF == 0 and n_chunks >= 2 * NBUF
    assert S % rows_per_w == 0

    w_per_b = S // rows_per_w

    mesh = plsc.VectorSubcoreMesh(core_axis_name="c", subcore_axis_name="s")

    @functools.partial(
        pl.kernel,
        mesh=mesh,
        out_type=jax.ShapeDtypeStruct((B, S, D), jnp.float32),
        scratch_types=[
            pltpu.VMEM((rows_per_w,), jnp.int32),
            pltpu.VMEM((NBUF, C, D), jnp.float32),
            pltpu.SemaphoreType.DMA((NBUF,)),
            pltpu.SemaphoreType.DMA((NBUF,)),
        ],
    )
    def gather_kernel(table_hbm, idx_hbm, out_hbm, idx_v, rows_v, gsem, wsem):
        wid = lax.axis_index("s") * NC + lax.axis_index("c")
        bi = wid // w_per_b
        base = (wid % w_per_b) * rows_per_w

        def gather_start(j, b):
            pltpu.async_copy(
                table_hbm.at[idx_v.at[pl.ds(j * C, C)]], rows_v.at[b],
                gsem.at[b])

        def gather_wait(j, b):
            pltpu.make_async_copy(
                table_hbm.at[idx_v.at[pl.ds(j * C, C)]], rows_v.at[b],
                gsem.at[b]).wait()

        def write_start(j, b):
            pltpu.async_copy(
                rows_v.at[b], out_hbm.at[bi, pl.ds(base + j * C, C)],
                wsem.at[b])

        def write_wait(j, b):
            pltpu.make_async_copy(
                rows_v.at[b], out_hbm.at[bi, pl.ds(base + j * C, C)],
                wsem.at[b]).wait()

        pltpu.sync_copy(idx_hbm.at[bi, pl.ds(base, rows_per_w)], idx_v)
        for b in range(NBUF):
            gather_start(b, b)
        def round_body(g):
            for b in range(NBUF):
                gather_wait(g + b, b)
                write_start(g + b, b)
            for b in range(NBUF):
                write_wait(g + b, b)
                gather_start(g + b + NBUF, b)
        pl.loop(0, n_chunks - NBUF, step=NBUF)(round_body)
        for b in range(NBUF):
            j = n_chunks - NBUF + b
            gather_wait(j, b)
            write_start(j, b)
        for b in range(NBUF):
            j = n_chunks - NBUF + b
            write_wait(j, b)

    return gather_kernel


def kernel(input_ids, word_embeddings):
    B, S = input_ids.shape
    V, D = word_embeddings.shape
    ids = input_ids.astype(jnp.int32)
    return _make_gather(B, S, V, D)(word_embeddings, ids)


# speedup vs baseline: 1.0486x; 1.0241x over previous
import functools

import jax
import jax.numpy as jnp
from jax import lax
from jax.experimental import pallas as pl
from jax.experimental.pallas import tpu as pltpu
from jax.experimental.pallas import tpu_sc as plsc


def _make_gather(B: int, S: int, V: int, D: int):
    info = plsc.get_sparse_core_info()
    NC, NS = info.num_cores, info.num_subcores
    NW = NC * NS
    N = B * S
    rows_per_w = N // NW
    C = 8
    NBUF = 8
    n_chunks = rows_per_w // C
    assert n_chunks % NBUF == 0 and n_chunks >= 2 * NBUF
    assert S % rows_per_w == 0

    w_per_b = S // rows_per_w

    mesh = plsc.VectorSubcoreMesh(core_axis_name="c", subcore_axis_name="s")

    @functools.partial(
        pl.kernel,
        mesh=mesh,
        out_type=jax.ShapeDtypeStruct((B, S, D), jnp.float32),
        scratch_types=[
            pltpu.VMEM((rows_per_w,), jnp.int32),
            pltpu.VMEM((NBUF, C, D), jnp.float32),
            pltpu.SemaphoreType.DMA((NBUF,)),
            pltpu.SemaphoreType.DMA((NBUF,)),
        ],
    )
    def gather_kernel(table_hbm, idx_hbm, out_hbm, idx_v, rows_v, gsem, wsem):
        wid = lax.axis_index("s") * NC + lax.axis_index("c")
        bi = wid // w_per_b
        base = (wid % w_per_b) * rows_per_w

        def gather_start(j, b):
            pltpu.async_copy(
                table_hbm.at[idx_v.at[pl.ds(j * C, C)]], rows_v.at[b],
                gsem.at[b])

        def gather_wait(j, b):
            pltpu.make_async_copy(
                table_hbm.at[idx_v.at[pl.ds(j * C, C)]], rows_v.at[b],
                gsem.at[b]).wait()

        def write_start(j, b):
            pltpu.async_copy(
                rows_v.at[b], out_hbm.at[bi, pl.ds(base + j * C, C)],
                wsem.at[b])

        def write_wait(j, b):
            pltpu.make_async_copy(
                rows_v.at[b], out_hbm.at[bi, pl.ds(base + j * C, C)],
                wsem.at[b]).wait()

        pltpu.sync_copy(idx_hbm.at[bi, pl.ds(base, rows_per_w)], idx_v)
        for b in range(NBUF):
            gather_start(b, b)
        def round_body(g):
            for b in range(NBUF):
                gather_wait(g + b, b)
                write_start(g + b, b)
            for b in range(NBUF):
                write_wait(g + b, b)
                gather_start(g + b + NBUF, b)
        pl.loop(0, n_chunks - NBUF, step=NBUF)(round_body)
        for b in range(NBUF):
            j = n_chunks - NBUF + b
            gather_wait(j, b)
            write_start(j, b)
        for b in range(NBUF):
            j = n_chunks - NBUF + b
            write_wait(j, b)

    return gather_kernel


def kernel(input_ids, word_embeddings):
    B, S = input_ids.shape
    V, D = word_embeddings.shape
    ids = input_ids.astype(jnp.int32)
    return _make_gather(B, S, V, D)(word_embeddings, ids)
